# baseline (speedup 1.0000x reference)
import functools

import jax
import jax.numpy as jnp
from jax import lax
from jax.experimental import pallas as pl
from jax.experimental.pallas import tpu as pltpu
from jax.experimental.pallas import tpu_sc as plsc

VOCAB = 100000
EMBED = 300
BATCH = 16384

_NUM_CORES = 2
_NUM_SUBCORES = 16
_NUM_WORKERS = _NUM_CORES * _NUM_SUBCORES
_ROWS_PER_WORKER = BATCH // _NUM_WORKERS
_CHUNK = 128
_NUM_CHUNKS = _ROWS_PER_WORKER // _CHUNK

_mesh = plsc.VectorSubcoreMesh(core_axis_name="c", subcore_axis_name="s")


@functools.partial(
    pl.kernel,
    mesh=_mesh,
    out_type=jax.ShapeDtypeStruct((BATCH, EMBED), jnp.float32),
    scratch_types=[
        pltpu.VMEM((_CHUNK,), jnp.int32),
        pltpu.VMEM((_CHUNK,), jnp.int32),
        pltpu.VMEM((_CHUNK, EMBED), jnp.float32),
        pltpu.VMEM((_CHUNK, EMBED), jnp.float32),
        pltpu.SemaphoreType.DMA,
        pltpu.SemaphoreType.DMA,
    ],
    compiler_params=pltpu.CompilerParams(use_tc_tiling_on_sc=False),
)
def _gather_kernel(tok_hbm, table_hbm, out_hbm, idx0, idx1, rows0, rows1, sem0, sem1):
    wid = lax.axis_index("s") * _NUM_CORES + lax.axis_index("c")
    base = wid * _ROWS_PER_WORKER
    idxs = (idx0, idx1)
    bufs = (rows0, rows1)
    sems = (sem0, sem1)
    for c in range(_NUM_CHUNKS):
        s = c % 2
        pltpu.sync_copy(tok_hbm.at[pl.ds(base + c * _CHUNK, _CHUNK)], idxs[s])
        cp = pltpu.async_copy(table_hbm.at[idxs[s]], bufs[s], sems[s])
        cp.wait()
        pltpu.sync_copy(bufs[s], out_hbm.at[pl.ds(base + c * _CHUNK, _CHUNK)])


def kernel(tokens, table):
    return _gather_kernel(tokens.astype(jnp.int32), table)
